# baseline (speedup 1.0000x reference)
import jax
import jax.numpy as jnp
from jax.experimental import pallas as pl
from jax.experimental.pallas import tpu as pltpu

_NORM_CLAMP = 1e-4


def _mmd_batch_kernel(x_ref, y_ref, w_ref, o_ref):
    x = x_ref[0]
    y = y_ref[0]
    w = w_ref[0, 0, 0]
    n = x.shape[0]
    d = x.shape[1]
    scale = -w / d

    def ksum(a, b):
        a2 = jnp.sum(a * a, axis=-1, keepdims=True)
        b2 = jnp.sum(b * b, axis=-1, keepdims=True)
        ab = jax.lax.dot_general(
            a, b, (((1,), (1,)), ((), ())),
            preferred_element_type=jnp.float32)
        sq = a2 + jnp.transpose(b2) - 2.0 * ab
        norm = jnp.sqrt(jnp.maximum(sq, 0.0))
        norm = jnp.maximum(norm, _NORM_CLAMP)
        return jnp.sum(jnp.exp(scale * norm))

    total = ksum(x, x) + ksum(y, y) - 2.0 * ksum(x, y)
    o_ref[0] = jnp.full((8, 128), total / (n * n), jnp.float32)


def kernel(x, y, w):
    B, N, D = x.shape
    per_batch = pl.pallas_call(
        _mmd_batch_kernel,
        grid=(B,),
        in_specs=[
            pl.BlockSpec((1, N, D), lambda b: (b, 0, 0)),
            pl.BlockSpec((1, N, D), lambda b: (b, 0, 0)),
            pl.BlockSpec((1, 1, 1), lambda b: (b, 0, 0)),
        ],
        out_specs=pl.BlockSpec((1, 8, 128), lambda b: (b, 0, 0)),
        out_shape=jax.ShapeDtypeStruct((B, 8, 128), jnp.float32),
        compiler_params=pltpu.CompilerParams(
            dimension_semantics=("parallel",)),
    )(x, y, w)
    return jnp.sum(per_batch[:, 0, 0]) / B


# speedup vs baseline: 1.2033x; 1.2033x over previous
import jax
import jax.numpy as jnp
from jax.experimental import pallas as pl
from jax.experimental.pallas import tpu as pltpu

_NORM_CLAMP = 1e-4


def _mmd_halfbatch_kernel(x_ref, y_ref, w_ref, o_ref):
    bpi = x_ref.shape[0]
    d = x_ref.shape[2]

    def batch_total(i):
        c = w_ref[i, 0, 0] / d
        xs = x_ref[i] * c
        ys = y_ref[i] * c
        eps2 = (_NORM_CLAMP * c) * (_NORM_CLAMP * c)

        def ksum(a, a2, b, b2):
            ab = jax.lax.dot_general(
                a, b, (((1,), (1,)), ((), ())),
                preferred_element_type=jnp.float32)
            sq = jnp.maximum(a2 + jnp.transpose(b2) - 2.0 * ab, eps2)
            norm = sq * jax.lax.rsqrt(sq)
            return jnp.sum(jnp.exp(-norm))

        x2 = jnp.sum(xs * xs, axis=-1, keepdims=True)
        y2 = jnp.sum(ys * ys, axis=-1, keepdims=True)
        return (ksum(xs, x2, xs, x2) + ksum(ys, y2, ys, y2)
                - 2.0 * ksum(xs, x2, ys, y2))

    acc = batch_total(0)
    for i in range(1, bpi):
        acc = acc + batch_total(i)
    o_ref[0] = jnp.full((8, 128), acc, jnp.float32)


def kernel(x, y, w):
    B, N, D = x.shape
    ncores = 2
    bpi = B // ncores
    partial = pl.pallas_call(
        _mmd_halfbatch_kernel,
        grid=(ncores,),
        in_specs=[
            pl.BlockSpec((bpi, N, D), lambda g: (g, 0, 0)),
            pl.BlockSpec((bpi, N, D), lambda g: (g, 0, 0)),
            pl.BlockSpec((bpi, 1, 1), lambda g: (g, 0, 0)),
        ],
        out_specs=pl.BlockSpec((1, 8, 128), lambda g: (g, 0, 0)),
        out_shape=jax.ShapeDtypeStruct((ncores, 8, 128), jnp.float32),
        compiler_params=pltpu.CompilerParams(
            dimension_semantics=("parallel",)),
    )(x, y, w)
    return jnp.sum(partial[:, 0, 0]) / (B * N * N)


# speedup vs baseline: 1.3817x; 1.1483x over previous
import jax
import jax.numpy as jnp
from jax.experimental import pallas as pl
from jax.experimental.pallas import tpu as pltpu

_NORM_CLAMP = 1e-4


def _mmd_kernel(x_ref, y_ref, w_ref, o_ref, acc_ref):
    step = pl.program_id(0)
    nsteps = pl.num_programs(0)
    bpi = x_ref.shape[0]
    n = x_ref.shape[1]
    d = x_ref.shape[2]

    @pl.when(step == 0)
    def _():
        acc_ref[0] = 0.0

    def batch_total(i):
        c = w_ref[i, 0, 0] / d
        xs = x_ref[i] * c
        ys = y_ref[i] * c
        eps2 = (_NORM_CLAMP * c) * (_NORM_CLAMP * c)
        ones = jnp.ones((n, 1), jnp.float32)

        x2 = jnp.sum(xs * xs, axis=-1, keepdims=True)
        y2 = jnp.sum(ys * ys, axis=-1, keepdims=True)
        ax = jnp.concatenate([xs, x2, ones], axis=1)
        ay = jnp.concatenate([ys, y2, ones], axis=1)
        bx = jnp.concatenate([-2.0 * xs, ones, x2], axis=1)
        by = jnp.concatenate([-2.0 * ys, ones, y2], axis=1)

        def ksum(a, b):
            sq = jax.lax.dot_general(
                a, b, (((1,), (1,)), ((), ())),
                preferred_element_type=jnp.float32)
            sq = jax.lax.max(sq, eps2)
            arg = (sq * jax.lax.rsqrt(sq)) * jnp.float32(-1.4426950408889634)
            return jnp.sum(jnp.exp2(arg))

        return ksum(ax, bx) + ksum(ay, by) - 2.0 * ksum(ax, by)

    total = batch_total(0)
    for i in range(1, bpi):
        total = total + batch_total(i)
    acc_ref[0] = acc_ref[0] + total

    @pl.when(step == nsteps - 1)
    def _():
        nb = nsteps * bpi
        o_ref[0] = acc_ref[0] / (nb * n * n)


def kernel(x, y, w):
    B, N, D = x.shape
    nsteps = 8
    bpi = B // nsteps
    out = pl.pallas_call(
        _mmd_kernel,
        grid=(nsteps,),
        in_specs=[
            pl.BlockSpec((bpi, N, D), lambda g: (g, 0, 0)),
            pl.BlockSpec((bpi, N, D), lambda g: (g, 0, 0)),
            pl.BlockSpec((bpi, 1, 1), lambda g: (g, 0, 0)),
        ],
        out_specs=pl.BlockSpec(memory_space=pltpu.SMEM),
        out_shape=jax.ShapeDtypeStruct((1,), jnp.float32),
        scratch_shapes=[pltpu.SMEM((1,), jnp.float32)],
        compiler_params=pltpu.CompilerParams(
            dimension_semantics=("arbitrary",)),
    )(x, y, w)
    return out.reshape(())


# speedup vs baseline: 1.3844x; 1.0020x over previous
import jax
import jax.numpy as jnp
from jax.experimental import pallas as pl
from jax.experimental.pallas import tpu as pltpu

_NORM_CLAMP = 1e-4
_NEG_LOG2E = -1.4426950408889634


def _mmd_kernel(x_ref, y_ref, w_ref, o_ref, acc_ref):
    step = pl.program_id(0)
    nsteps = pl.num_programs(0)
    bpi = x_ref.shape[0]
    n = x_ref.shape[1]
    d = x_ref.shape[2]

    @pl.when(step == 0)
    def _():
        acc_ref[0] = 0.0

    def batch_total(i):
        c = w_ref[i, 0, 0] / d
        xs = x_ref[i] * c
        ys = y_ref[i] * c
        xm2 = -2.0 * xs
        eps2 = (_NORM_CLAMP * c) * (_NORM_CLAMP * c)

        x2 = jnp.sum(xs * xs, axis=-1, keepdims=True)
        y2 = jnp.sum(ys * ys, axis=-1, keepdims=True)

        def ksum(a2col, b2row, am2, b):
            ab = jax.lax.dot_general(
                am2, b, (((1,), (1,)), ((), ())),
                preferred_element_type=jnp.float32)
            sq = jax.lax.max(a2col + (b2row + ab), eps2)
            arg = (sq * jax.lax.rsqrt(sq)) * jnp.float32(_NEG_LOG2E)
            return jnp.sum(jnp.exp2(arg))

        x2row = jnp.transpose(x2)
        y2row = jnp.transpose(y2)
        return (ksum(x2, x2row, xm2, xs) + ksum(y2, y2row, -2.0 * ys, ys)
                - 2.0 * ksum(x2, y2row, xm2, ys))

    total = batch_total(0)
    for i in range(1, bpi):
        total = total + batch_total(i)
    acc_ref[0] = acc_ref[0] + total

    @pl.when(step == nsteps - 1)
    def _():
        nb = nsteps * bpi
        o_ref[0] = acc_ref[0] / (nb * n * n)


def kernel(x, y, w):
    B, N, D = x.shape
    nsteps = 8
    bpi = B // nsteps
    out = pl.pallas_call(
        _mmd_kernel,
        grid=(nsteps,),
        in_specs=[
            pl.BlockSpec((bpi, N, D), lambda g: (g, 0, 0)),
            pl.BlockSpec((bpi, N, D), lambda g: (g, 0, 0)),
            pl.BlockSpec((bpi, 1, 1), lambda g: (g, 0, 0)),
        ],
        out_specs=pl.BlockSpec(memory_space=pltpu.SMEM),
        out_shape=jax.ShapeDtypeStruct((1,), jnp.float32),
        scratch_shapes=[pltpu.SMEM((1,), jnp.float32)],
        compiler_params=pltpu.CompilerParams(
            dimension_semantics=("arbitrary",)),
    )(x, y, w)
    return out.reshape(())


# speedup vs baseline: 1.6988x; 1.2271x over previous
import jax
import jax.numpy as jnp
from jax.experimental import pallas as pl
from jax.experimental.pallas import tpu as pltpu

_NORM_CLAMP = 1e-4
_NEG_LOG2E = -1.4426950408889634


def _dot_t(a, b):
    return jax.lax.dot_general(a, b, (((1,), (1,)), ((), ())),
                               preferred_element_type=jnp.float32)


def _mmd_kernel(x_ref, y_ref, w_ref, o_ref, accp_ref, accx_ref):
    step = pl.program_id(0)
    nsteps = pl.num_programs(0)
    bpi = x_ref.shape[0]
    n = x_ref.shape[1]
    d = x_ref.shape[2]
    ones_1d = jnp.ones((1, d), jnp.float32)

    @pl.when(step == 0)
    def _():
        accp_ref[...] = jnp.zeros_like(accp_ref)
        accx_ref[...] = jnp.zeros_like(accx_ref)

    def batch_tiles(i):
        c = w_ref[i, 0, 0] / d
        xs = x_ref[i] * c
        ys = y_ref[i] * c
        xm2 = -2.0 * xs
        ym2 = -2.0 * ys
        eps2 = (_NORM_CLAMP * c) * (_NORM_CLAMP * c)

        xx = xs * xs
        yy = ys * ys
        x2 = _dot_t(xx, ones_1d)
        y2 = _dot_t(yy, ones_1d)
        x2row = _dot_t(ones_1d, xx)
        y2row = _dot_t(ones_1d, yy)

        def ktile(a2col, b2row, am2, b):
            ab = _dot_t(am2, b)
            sq = jax.lax.max(a2col + (b2row + ab), eps2)
            arg = (sq * jax.lax.rsqrt(sq)) * jnp.float32(_NEG_LOG2E)
            return jnp.exp2(arg)

        pos = ktile(x2, x2row, xm2, xs) + ktile(y2, y2row, ym2, ys)
        xy = ktile(x2, y2row, xm2, ys)
        return pos, xy

    pos, xy = batch_tiles(0)
    for i in range(1, bpi):
        p2, x2t = batch_tiles(i)
        pos = pos + p2
        xy = xy + x2t
    accp_ref[...] = accp_ref[...] + pos
    accx_ref[...] = accx_ref[...] + xy

    @pl.when(step == nsteps - 1)
    def _():
        nb = nsteps * bpi
        total = jnp.sum(accp_ref[...]) - 2.0 * jnp.sum(accx_ref[...])
        o_ref[0] = total / (nb * n * n)


def kernel(x, y, w):
    B, N, D = x.shape
    nsteps = 16
    bpi = B // nsteps
    out = pl.pallas_call(
        _mmd_kernel,
        grid=(nsteps,),
        in_specs=[
            pl.BlockSpec((bpi, N, D), lambda g: (g, 0, 0)),
            pl.BlockSpec((bpi, N, D), lambda g: (g, 0, 0)),
            pl.BlockSpec((bpi, 1, 1), lambda g: (g, 0, 0)),
        ],
        out_specs=pl.BlockSpec(memory_space=pltpu.SMEM),
        out_shape=jax.ShapeDtypeStruct((1,), jnp.float32),
        scratch_shapes=[pltpu.VMEM((N, N), jnp.float32),
                        pltpu.VMEM((N, N), jnp.float32)],
        compiler_params=pltpu.CompilerParams(
            dimension_semantics=("arbitrary",)),
    )(x, y, w)
    return out.reshape(())


# speedup vs baseline: 2.0156x; 1.1864x over previous
import jax
import jax.numpy as jnp
from jax.experimental import pallas as pl
from jax.experimental.pallas import tpu as pltpu

_NORM_CLAMP = 1e-4
_NEG_LOG2E = -1.4426950408889634


def _dot_t(a, b):
    return jax.lax.dot_general(a, b, (((1,), (1,)), ((), ())),
                               preferred_element_type=jnp.float32)


def _mmd_kernel(x_ref, y_ref, w_ref, o_ref):
    nb = x_ref.shape[0]
    n = x_ref.shape[1]
    d = x_ref.shape[2]
    h = n // 2
    ones_1d = jnp.ones((1, d), jnp.float32)

    def batch_tiles(i):
        c = w_ref[i, 0, 0] / d
        xs = x_ref[i] * c
        ys = y_ref[i] * c
        xm2 = -2.0 * xs
        ym2 = -2.0 * ys
        eps2 = (_NORM_CLAMP * c) * (_NORM_CLAMP * c)

        xx = xs * xs
        yy = ys * ys
        x2 = _dot_t(xx, ones_1d)
        y2 = _dot_t(yy, ones_1d)
        x2row = _dot_t(ones_1d, xx)
        y2row = _dot_t(ones_1d, yy)

        def ktile(a2col, b2row, am2, b):
            ab = _dot_t(am2, b)
            sq = jax.lax.max(a2col + (b2row + ab), eps2)
            arg = (sq * jax.lax.rsqrt(sq)) * jnp.float32(_NEG_LOG2E)
            return jnp.exp2(arg)

        def sym_quads(a2, a2row, am2, a):
            q11 = ktile(a2[:h], a2row[:, :h], am2[:h], a[:h])
            q22 = ktile(a2[h:], a2row[:, h:], am2[h:], a[h:])
            q12 = ktile(a2[:h], a2row[:, h:], am2[:h], a[h:])
            return q11 + q22, q12

        w1x, w2x = sym_quads(x2, x2row, xm2, xs)
        w1y, w2y = sym_quads(y2, y2row, ym2, ys)
        kxy = ktile(x2, y2row, xm2, ys)
        return w1x + w1y, w2x + w2y, kxy

    t1, t2, tx = batch_tiles(0)
    for i in range(1, nb):
        u1, u2, ux = batch_tiles(i)
        t1 = t1 + u1
        t2 = t2 + u2
        tx = tx + ux

    total = jnp.sum(t1) + 2.0 * jnp.sum(t2) - 2.0 * jnp.sum(tx)
    o_ref[0] = total / (nb * n * n)


def kernel(x, y, w):
    B, N, D = x.shape
    out = pl.pallas_call(
        _mmd_kernel,
        in_specs=[
            pl.BlockSpec(memory_space=pltpu.VMEM),
            pl.BlockSpec(memory_space=pltpu.VMEM),
            pl.BlockSpec(memory_space=pltpu.VMEM),
        ],
        out_specs=pl.BlockSpec(memory_space=pltpu.SMEM),
        out_shape=jax.ShapeDtypeStruct((1,), jnp.float32),
    )(x, y, w)
    return out.reshape(())
